# Initial kernel scaffold; baseline (speedup 1.0000x reference)
import jax
import jax.numpy as jnp
from jax.experimental import pallas as pl


def kernel(features, rois):
    raise NotImplementedError("write your pallas kernel here")


# speedup vs baseline: 16.1495x; 16.1495x over previous
import functools

import jax
import jax.numpy as jnp
from jax import lax
from jax.experimental import pallas as pl
from jax.experimental.pallas import tpu as pltpu
from jax.experimental.pallas import tpu_sc as plsc

OUT_H = 7
OUT_W = 7
SCALE = 0.0625
NEG = -3.4e38
R7 = 0.14285714924335479736328125

N_IMG = 2
C = 256
H = 50
W = 50
K = 1000
REG = 12
NCHUNK = C // 16


def _roipool_sc_body(tbl_hbm, rois_hbm, out_hbm, region, outbuf, rois_v, sem):
  nc = 2
  wid = lax.axis_index("s") * nc + lax.axis_index("c")
  start = 31 * wid + jnp.minimum(wid, 8)
  cnt = jnp.where(wid < 8, 32, 31)
  pltpu.sync_copy(rois_hbm.at[pl.ds(start * 16, 32 * 16)], rois_v)

  def fl7(v):
    q = v.astype(jnp.float32) * jnp.float32(R7)
    t = q.astype(jnp.int32)
    return t - (t.astype(jnp.float32) > q).astype(jnp.int32)

  def ce7(v):
    q = v.astype(jnp.float32) * jnp.float32(R7)
    t = q.astype(jnp.int32)
    return t + (t.astype(jnp.float32) < q).astype(jnp.int32)

  def do_roi(r, carry):
    vec = rois_v[pl.ds(r * 16, 16)]
    b = vec[0].astype(jnp.int32)
    x1 = (vec[1] * SCALE).astype(jnp.int32)
    y1 = (vec[2] * SCALE).astype(jnp.int32)
    x2 = (vec[3] * SCALE).astype(jnp.int32)
    y2 = (vec[4] * SCALE).astype(jnp.int32)
    rw = jnp.maximum(x2 - x1 + 1, 1)
    rh = jnp.maximum(y2 - y1 + 1, 1)
    yoff = jnp.minimum(y1, H - REG)
    xoff = jnp.minimum(x1, W - REG)
    row0 = b * H + yoff
    copies = [
        pltpu.async_copy(
            tbl_hbm.at[pl.ds(((row0 + dy) * W + xoff) * C, REG * C)],
            region.at[dy], sem)
        for dy in range(REG)
    ]
    for cp in copies:
      cp.wait()

    def do_i(i, carry):
      hs = jnp.minimum(fl7(i * rh) + y1, H)
      he = jnp.minimum(ce7((i + 1) * rh) + y1, H)
      cy = he - hs
      ry0 = hs - yoff

      def do_j(j, carry):
        ws = jnp.minimum(fl7(j * rw) + x1, W)
        we = jnp.minimum(ce7((j + 1) * rw) + x1, W)
        cx = we - ws
        rx0 = ws - xoff

        def do_dy(dy, acc):
          def do_dx(dx, acc):
            return tuple(
                jnp.maximum(acc[ch],
                            region[ry0 + dy,
                                   pl.ds((rx0 + dx) * C + ch * 16, 16)])
                for ch in range(NCHUNK))
          return lax.fori_loop(0, jnp.maximum(cx, 0), do_dx, acc)

        acc0 = tuple(jnp.full((16,), NEG, jnp.float32) for _ in range(NCHUNK))
        acc = lax.fori_loop(0, jnp.maximum(cy, 0), do_dy, acc0)
        valid = (cy > 0) & (cx > 0)
        lane = lax.iota(jnp.int32, 16)
        cell = i * OUT_W + j
        idx0 = lane * (OUT_H * OUT_W) + cell
        for ch in range(NCHUNK):
          val = jnp.where(valid, acc[ch], jnp.float32(0.0))
          plsc.store_scatter(outbuf, [idx0 + ch * 16 * OUT_H * OUT_W], val)
        return carry

      return lax.fori_loop(0, OUT_W, do_j, carry)

    lax.fori_loop(0, OUT_H, do_i, 0)
    k = start + r
    pltpu.sync_copy(outbuf, out_hbm.at[pl.ds(k * (C * OUT_H * OUT_W),
                                             C * OUT_H * OUT_W)])
    return carry

  lax.fori_loop(0, cnt, do_roi, 0)


@jax.jit
def kernel(features, rois):
  tbl = jnp.transpose(features, (0, 2, 3, 1)).reshape(N_IMG * H * W * C)
  rois16 = (jnp.zeros((K + 8, 16), jnp.float32).at[:K, :5].set(rois)
            .reshape((K + 8) * 16))
  mesh = plsc.VectorSubcoreMesh(core_axis_name="c", subcore_axis_name="s")
  fn = pl.kernel(
      _roipool_sc_body,
      out_type=jax.ShapeDtypeStruct((K * C * OUT_H * OUT_W,), jnp.float32),
      mesh=mesh,
      compiler_params=pltpu.CompilerParams(needs_layout_passes=False),
      scratch_types=[
          pltpu.VMEM((REG, REG * C), jnp.float32),
          pltpu.VMEM((C * OUT_H * OUT_W,), jnp.float32),
          pltpu.VMEM((32 * 16,), jnp.float32),
          pltpu.SemaphoreType.DMA,
      ],
  )
  return fn(tbl, rois16).reshape(K, C, OUT_H, OUT_W)
